# baseline (speedup 1.0000x reference)
import functools

import jax
import jax.numpy as jnp
from jax import lax
from jax.experimental import pallas as pl
from jax.experimental.pallas import tpu as pltpu
from jax.experimental.pallas import tpu_sc as plsc

N = 10000
E = 320000
H = 128
C = 10
G = 64

NC = 2
NS = 16
K = 88
SUPER = 8
NSUP = 29
CHUNKS = SUPER * NSUP
E_PAD = NS * CHUNKS * K
NBUF = 4
SKEW = 3
ACC_ROWS = 10112
ZROWS = 632

BLK = 1000
NB = N // BLK


def _segsum_body(*refs):
    (xs_hbm, src_hbm, dst_hbm, zeros_hbm, out_hbm) = refs[:5]
    (idxs_v, idxd_v) = refs[5:7]
    rows = refs[7:7 + NBUF]
    acc = refs[7 + NBUF]
    gsem = refs[8 + NBUF:8 + 2 * NBUF]
    ssem = refs[8 + 2 * NBUF:8 + 3 * NBUF]
    c = lax.axis_index("c")
    t = lax.axis_index("s")
    pltpu.sync_copy(zeros_hbm, acc.at[pl.ds(t * ZROWS, ZROWS)])
    plsc.subcore_barrier()

    def superchunk(s, carry):
        pltpu.sync_copy(src_hbm.at[c, t, pl.ds(s * SUPER, SUPER)], idxs_v)
        pltpu.sync_copy(dst_hbm.at[c, t, pl.ds(s * SUPER, SUPER)], idxd_v)
        gd = [None] * NBUF
        sd = [None] * NBUF
        for jj in range(SUPER):
            b = jj % NBUF
            if sd[b] is not None:
                sd[b].wait()
            gd[b] = pltpu.async_copy(xs_hbm.at[idxs_v.at[jj]], rows[b], gsem[b])
            if jj >= SKEW:
                pb = (jj - SKEW) % NBUF
                gd[pb].wait()
                sd[pb] = pltpu.async_copy(rows[pb], acc.at[idxd_v.at[jj - SKEW]],
                                          ssem[pb], add=True)
        for jj in range(SUPER - SKEW, SUPER):
            pb = jj % NBUF
            gd[pb].wait()
            sd[pb] = pltpu.async_copy(rows[pb], acc.at[idxd_v.at[jj]],
                                      ssem[pb], add=True)
        for b in range(NBUF):
            if sd[b] is not None:
                sd[b].wait()
        return carry

    lax.fori_loop(0, NSUP, superchunk, 0)
    plsc.subcore_barrier()
    pltpu.sync_copy(acc.at[pl.ds(t * ZROWS, ZROWS)],
                    out_hbm.at[c, pl.ds(t * ZROWS, ZROWS)])


@functools.cache
def _make_segsum():
    return pl.kernel(
        _segsum_body,
        out_type=jax.ShapeDtypeStruct((NC, ACC_ROWS, H), jnp.float32),
        mesh=plsc.VectorSubcoreMesh(core_axis_name="c", subcore_axis_name="s",
                                    num_cores=NC, num_subcores=NS),
        scratch_types=[
            pltpu.VMEM((SUPER, K), jnp.int32),
            pltpu.VMEM((SUPER, K), jnp.int32),
        ] + [pltpu.VMEM((K, H), jnp.float32)] * NBUF + [
            pltpu.VMEM_SHARED((ACC_ROWS, H), jnp.float32),
        ] + [pltpu.SemaphoreType.DMA] * (2 * NBUF),
    )


def _mlp_body(agg_ref, h_ref, ep_ref, w1_ref, b1_ref, w2_ref, b2_ref,
              h2_ref, s1_ref, s2_ref):
    j = pl.program_id(1)
    out = agg_ref[0] + ep_ref[0] * h_ref[0]
    z = jnp.maximum(jnp.dot(out, w1_ref[0],
                            preferred_element_type=jnp.float32) + b1_ref[0], 0.0)
    z = jnp.maximum(jnp.dot(z, w2_ref[0],
                            preferred_element_type=jnp.float32) + b2_ref[0], 0.0)
    h2_ref[0] = z
    cs = jnp.sum(z, axis=0, keepdims=True)
    cq = jnp.sum(z * z, axis=0, keepdims=True)

    @pl.when(j == 0)
    def _():
        s1_ref[0] = cs
        s2_ref[0] = cq

    @pl.when(j > 0)
    def _():
        s1_ref[0] += cs
        s2_ref[0] += cq


_row = pl.BlockSpec((1, BLK, H), lambda i, j: (i, j, 0))
_per_ens = pl.BlockSpec((1, 1, H), lambda i, j: (i, 0, 0))
_wspec = pl.BlockSpec((1, H, H), lambda i, j: (i, 0, 0))

_mlp_call = pl.pallas_call(
    _mlp_body,
    grid=(NC, NB),
    in_specs=[_row, _row, _per_ens, _wspec, _per_ens, _wspec, _per_ens],
    out_specs=[_row, _per_ens, _per_ens],
    out_shape=[
        jax.ShapeDtypeStruct((NC, N, H), jnp.float32),
        jax.ShapeDtypeStruct((NC, 1, H), jnp.float32),
        jax.ShapeDtypeStruct((NC, 1, H), jnp.float32),
    ],
)


def _norm_body(h2_ref, s1_ref, s2_ref, g_ref, b_ref, o_ref):
    mu = s1_ref[0] * (1.0 / N)
    var = s2_ref[0] * (1.0 / N) - mu * mu
    inv = lax.rsqrt(var + 1e-5)
    o_ref[0] = jnp.maximum((h2_ref[0] - mu) * (inv * g_ref[0]) + b_ref[0], 0.0)


_norm_call = pl.pallas_call(
    _norm_body,
    grid=(NC, NB),
    in_specs=[_row, _per_ens, _per_ens, _per_ens, _per_ens],
    out_specs=_row,
    out_shape=jax.ShapeDtypeStruct((NC, N, H), jnp.float32),
)


def _pool_body(h2_ref, s1_ref, s2_ref, g_ref, b_ref, batch_ref,
               w1_ref, b1_ref, w2_ref, b2_ref, y_ref, pool_acc, cnt_acc):
    j = pl.program_id(1)
    mu = s1_ref[0] * (1.0 / N)
    var = s2_ref[0] * (1.0 / N) - mu * mu
    inv = lax.rsqrt(var + 1e-5)
    h = jnp.maximum((h2_ref[0] - mu) * (inv * g_ref[0]) + b_ref[0], 0.0)
    seg = batch_ref[0, 0, :]
    iota = lax.broadcasted_iota(jnp.int32, (G, BLK), 0)
    onehot = (iota == seg[None, :]).astype(jnp.float32)

    @pl.when(j == 0)
    def _():
        pool_acc[...] = jnp.zeros((G, H), jnp.float32)
        cnt_acc[...] = jnp.zeros((G, H), jnp.float32)

    pool_acc[...] += jnp.dot(onehot, h, preferred_element_type=jnp.float32)
    cnt_acc[...] += jnp.broadcast_to(
        jnp.sum(onehot, axis=1, keepdims=True), (G, H))

    @pl.when(j == NB - 1)
    def _():
        mean = pool_acc[...] / jnp.maximum(cnt_acc[...], 1.0)
        z = jnp.maximum(jnp.dot(mean, w1_ref[0],
                                preferred_element_type=jnp.float32) + b1_ref[0], 0.0)
        y_ref[0] = jnp.dot(z, w2_ref[0],
                           preferred_element_type=jnp.float32) + b2_ref[0]


_pool_call = pl.pallas_call(
    _pool_body,
    grid=(NC, NB),
    in_specs=[
        _row, _per_ens, _per_ens, _per_ens, _per_ens,
        pl.BlockSpec((1, 1, BLK), lambda i, j: (i * NB + j, 0, 0)),
        _wspec, _per_ens,
        pl.BlockSpec((1, H, C), lambda i, j: (i, 0, 0)),
        pl.BlockSpec((1, 1, C), lambda i, j: (i, 0, 0)),
    ],
    out_specs=pl.BlockSpec((1, G, C), lambda i, j: (i, 0, 0)),
    out_shape=jax.ShapeDtypeStruct((NC, G, C), jnp.float32),
    scratch_shapes=[
        pltpu.VMEM((G, H), jnp.float32),
        pltpu.VMEM((G, H), jnp.float32),
    ],
)


def _prep_edges(ei, ens):
    src = jnp.pad(ei[0], (0, E_PAD - E)) + ens * N
    dst = jnp.pad(ei[1], (0, E_PAD - E), constant_values=N)
    return (src.reshape(NS, CHUNKS, K).astype(jnp.int32),
            dst.reshape(NS, CHUNKS, K).astype(jnp.int32))


def _stack_mlp(ps):
    out = {}
    for k in ("W1", "b1", "W2", "b2", "gamma", "beta"):
        out[k] = jnp.stack([p[k] for p in ps])
    out["b1"] = out["b1"].reshape(NC, 1, H)
    out["b2"] = out["b2"].reshape(NC, 1, H)
    out["gamma"] = out["gamma"].reshape(NC, 1, H)
    out["beta"] = out["beta"].reshape(NC, 1, H)
    out["ep"] = jnp.broadcast_to(
        jnp.stack([1.0 + p["eps"] for p in ps]).reshape(NC, 1, 1), (NC, 1, H))
    return out


def kernel(x0, x1, edge_index0, edge_index1, batch0, batch1, params):
    xs = jnp.concatenate([x0, x1], axis=0)
    s0, d0 = _prep_edges(edge_index0, 0)
    s1, d1 = _prep_edges(edge_index1, 1)
    src_idx = jnp.stack([s0, s1])
    dst_idx = jnp.stack([d0, d1])
    zeros = jnp.zeros((ZROWS, H), jnp.float32)
    batch = jnp.stack([batch0, batch1]).reshape(NC * NB, 1, BLK)

    layer_ps = [
        _stack_mlp([params["conv1"][0], params["conv1"][1]]),
        _stack_mlp([params["convs"][0], params["convs"][0]]),
        _stack_mlp([params["convs"][1], params["convs"][1]]),
    ]

    h = xs
    h2 = cs = cq = None
    for l, p in enumerate(layer_ps):
        agg = _make_segsum()(h, src_idx, dst_idx, zeros)
        h2, cs, cq = _mlp_call(agg, h.reshape(NC, N, H), p["ep"],
                               p["W1"], p["b1"], p["W2"], p["b2"])
        if l < 2:
            h = _norm_call(h2, cs, cq, p["gamma"], p["beta"]).reshape(NC * N, H)

    p3 = layer_ps[2]
    lin1W = jnp.stack([params["lin1"][0]["W"], params["lin1"][1]["W"]])
    lin1b = jnp.stack([params["lin1"][0]["b"], params["lin1"][1]["b"]]).reshape(NC, 1, H)
    lin2W = jnp.stack([params["lin2"][0]["W"], params["lin2"][1]["W"]])
    lin2b = jnp.stack([params["lin2"][0]["b"], params["lin2"][1]["b"]]).reshape(NC, 1, C)
    y = _pool_call(h2, cs, cq, p3["gamma"], p3["beta"], batch,
                   lin1W, lin1b, lin2W, lin2b)
    return y


# speedup vs baseline: 1.3837x; 1.3837x over previous
import functools

import jax
import jax.numpy as jnp
from jax import lax
from jax.experimental import pallas as pl
from jax.experimental.pallas import tpu as pltpu
from jax.experimental.pallas import tpu_sc as plsc

N = 10000
E = 320000
H = 128
C = 10
G = 64

NC = 2
NS = 16
K = 120
SUPER = 8
NSUP = 21
CHUNKS = SUPER * NSUP
E_PAD = NS * CHUNKS * K
NBUF = 3
SKEW = 2
ACC_ROWS = 10112
ZROWS = 632

BLK = 1000
NB = N // BLK


def _segsum_body(*refs):
    (xs_hbm, src_hbm, dst_hbm, zeros_hbm, out_hbm) = refs[:5]
    (idxs_v, idxd_v) = refs[5:7]
    rows = refs[7:7 + NBUF]
    acc = refs[7 + NBUF]
    gsem = refs[8 + NBUF:8 + 2 * NBUF]
    ssem = refs[8 + 2 * NBUF:8 + 3 * NBUF]
    c = lax.axis_index("c")
    t = lax.axis_index("s")
    pltpu.sync_copy(zeros_hbm, acc.at[pl.ds(t * ZROWS, ZROWS)])
    plsc.subcore_barrier()

    def superchunk(s, carry):
        pltpu.sync_copy(src_hbm.at[c, t, pl.ds(s * SUPER, SUPER)], idxs_v)
        pltpu.sync_copy(dst_hbm.at[c, t, pl.ds(s * SUPER, SUPER)], idxd_v)
        gd = [None] * NBUF
        sd = [None] * NBUF
        for jj in range(SUPER):
            b = jj % NBUF
            if sd[b] is not None:
                sd[b].wait()
            gd[b] = pltpu.async_copy(xs_hbm.at[idxs_v.at[jj]], rows[b], gsem[b])
            if jj >= SKEW:
                pb = (jj - SKEW) % NBUF
                gd[pb].wait()
                sd[pb] = pltpu.async_copy(rows[pb], acc.at[idxd_v.at[jj - SKEW]],
                                          ssem[pb], add=True)
        for jj in range(SUPER - SKEW, SUPER):
            pb = jj % NBUF
            gd[pb].wait()
            sd[pb] = pltpu.async_copy(rows[pb], acc.at[idxd_v.at[jj]],
                                      ssem[pb], add=True)
        for b in range(NBUF):
            if sd[b] is not None:
                sd[b].wait()
        return carry

    lax.fori_loop(0, NSUP, superchunk, 0)
    plsc.subcore_barrier()
    pltpu.sync_copy(acc.at[pl.ds(t * ZROWS, ZROWS)],
                    out_hbm.at[c, pl.ds(t * ZROWS, ZROWS)])


@functools.cache
def _make_segsum():
    return pl.kernel(
        _segsum_body,
        out_type=jax.ShapeDtypeStruct((NC, ACC_ROWS, H), jnp.float32),
        mesh=plsc.VectorSubcoreMesh(core_axis_name="c", subcore_axis_name="s",
                                    num_cores=NC, num_subcores=NS),
        scratch_types=[
            pltpu.VMEM((SUPER, K), jnp.int32),
            pltpu.VMEM((SUPER, K), jnp.int32),
        ] + [pltpu.VMEM((K, H), jnp.float32)] * NBUF + [
            pltpu.VMEM_SHARED((ACC_ROWS, H), jnp.float32),
        ] + [pltpu.SemaphoreType.DMA] * (2 * NBUF),
    )


def _mlp_body(agg_ref, h_ref, ep_ref, w1_ref, b1_ref, w2_ref, b2_ref,
              h2_ref, s1_ref, s2_ref):
    j = pl.program_id(1)
    out = agg_ref[0] + ep_ref[0] * h_ref[0]
    z = jnp.maximum(jnp.dot(out, w1_ref[0],
                            preferred_element_type=jnp.float32) + b1_ref[0], 0.0)
    z = jnp.maximum(jnp.dot(z, w2_ref[0],
                            preferred_element_type=jnp.float32) + b2_ref[0], 0.0)
    h2_ref[0] = z
    cs = jnp.sum(z, axis=0, keepdims=True)
    cq = jnp.sum(z * z, axis=0, keepdims=True)

    @pl.when(j == 0)
    def _():
        s1_ref[0] = cs
        s2_ref[0] = cq

    @pl.when(j > 0)
    def _():
        s1_ref[0] += cs
        s2_ref[0] += cq


_row = pl.BlockSpec((1, BLK, H), lambda i, j: (i, j, 0))
_per_ens = pl.BlockSpec((1, 1, H), lambda i, j: (i, 0, 0))
_wspec = pl.BlockSpec((1, H, H), lambda i, j: (i, 0, 0))

_mlp_call = pl.pallas_call(
    _mlp_body,
    grid=(NC, NB),
    in_specs=[_row, _row, _per_ens, _wspec, _per_ens, _wspec, _per_ens],
    out_specs=[_row, _per_ens, _per_ens],
    out_shape=[
        jax.ShapeDtypeStruct((NC, N, H), jnp.float32),
        jax.ShapeDtypeStruct((NC, 1, H), jnp.float32),
        jax.ShapeDtypeStruct((NC, 1, H), jnp.float32),
    ],
)


def _norm_body(h2_ref, s1_ref, s2_ref, g_ref, b_ref, o_ref):
    mu = s1_ref[0] * (1.0 / N)
    var = s2_ref[0] * (1.0 / N) - mu * mu
    inv = lax.rsqrt(var + 1e-5)
    o_ref[0] = jnp.maximum((h2_ref[0] - mu) * (inv * g_ref[0]) + b_ref[0], 0.0)


_norm_call = pl.pallas_call(
    _norm_body,
    grid=(NC, NB),
    in_specs=[_row, _per_ens, _per_ens, _per_ens, _per_ens],
    out_specs=_row,
    out_shape=jax.ShapeDtypeStruct((NC, N, H), jnp.float32),
)


def _pool_body(h2_ref, s1_ref, s2_ref, g_ref, b_ref, batch_ref,
               w1_ref, b1_ref, w2_ref, b2_ref, y_ref, pool_acc, cnt_acc):
    j = pl.program_id(1)
    mu = s1_ref[0] * (1.0 / N)
    var = s2_ref[0] * (1.0 / N) - mu * mu
    inv = lax.rsqrt(var + 1e-5)
    h = jnp.maximum((h2_ref[0] - mu) * (inv * g_ref[0]) + b_ref[0], 0.0)
    seg = batch_ref[0, 0, :]
    iota = lax.broadcasted_iota(jnp.int32, (G, BLK), 0)
    onehot = (iota == seg[None, :]).astype(jnp.float32)

    @pl.when(j == 0)
    def _():
        pool_acc[...] = jnp.zeros((G, H), jnp.float32)
        cnt_acc[...] = jnp.zeros((G, H), jnp.float32)

    pool_acc[...] += jnp.dot(onehot, h, preferred_element_type=jnp.float32)
    cnt_acc[...] += jnp.broadcast_to(
        jnp.sum(onehot, axis=1, keepdims=True), (G, H))

    @pl.when(j == NB - 1)
    def _():
        mean = pool_acc[...] / jnp.maximum(cnt_acc[...], 1.0)
        z = jnp.maximum(jnp.dot(mean, w1_ref[0],
                                preferred_element_type=jnp.float32) + b1_ref[0], 0.0)
        y_ref[0] = jnp.dot(z, w2_ref[0],
                           preferred_element_type=jnp.float32) + b2_ref[0]


_pool_call = pl.pallas_call(
    _pool_body,
    grid=(NC, NB),
    in_specs=[
        _row, _per_ens, _per_ens, _per_ens, _per_ens,
        pl.BlockSpec((1, 1, BLK), lambda i, j: (i * NB + j, 0, 0)),
        _wspec, _per_ens,
        pl.BlockSpec((1, H, C), lambda i, j: (i, 0, 0)),
        pl.BlockSpec((1, 1, C), lambda i, j: (i, 0, 0)),
    ],
    out_specs=pl.BlockSpec((1, G, C), lambda i, j: (i, 0, 0)),
    out_shape=jax.ShapeDtypeStruct((NC, G, C), jnp.float32),
    scratch_shapes=[
        pltpu.VMEM((G, H), jnp.float32),
        pltpu.VMEM((G, H), jnp.float32),
    ],
)


def _prep_edges(ei, ens):
    src = jnp.pad(ei[0], (0, E_PAD - E)) + ens * N
    dst = jnp.pad(ei[1], (0, E_PAD - E), constant_values=N)
    return (src.reshape(NS, CHUNKS, K).astype(jnp.int32),
            dst.reshape(NS, CHUNKS, K).astype(jnp.int32))


def _stack_mlp(ps):
    out = {}
    for k in ("W1", "b1", "W2", "b2", "gamma", "beta"):
        out[k] = jnp.stack([p[k] for p in ps])
    out["b1"] = out["b1"].reshape(NC, 1, H)
    out["b2"] = out["b2"].reshape(NC, 1, H)
    out["gamma"] = out["gamma"].reshape(NC, 1, H)
    out["beta"] = out["beta"].reshape(NC, 1, H)
    out["ep"] = jnp.broadcast_to(
        jnp.stack([1.0 + p["eps"] for p in ps]).reshape(NC, 1, 1), (NC, 1, H))
    return out


def kernel(x0, x1, edge_index0, edge_index1, batch0, batch1, params):
    xs = jnp.concatenate([x0, x1], axis=0)
    s0, d0 = _prep_edges(edge_index0, 0)
    s1, d1 = _prep_edges(edge_index1, 1)
    src_idx = jnp.stack([s0, s1])
    dst_idx = jnp.stack([d0, d1])
    zeros = jnp.zeros((ZROWS, H), jnp.float32)
    batch = jnp.stack([batch0, batch1]).reshape(NC * NB, 1, BLK)

    layer_ps = [
        _stack_mlp([params["conv1"][0], params["conv1"][1]]),
        _stack_mlp([params["convs"][0], params["convs"][0]]),
        _stack_mlp([params["convs"][1], params["convs"][1]]),
    ]

    h = xs
    h2 = cs = cq = None
    for l, p in enumerate(layer_ps):
        agg = _make_segsum()(h, src_idx, dst_idx, zeros)
        h2, cs, cq = _mlp_call(agg, h.reshape(NC, N, H), p["ep"],
                               p["W1"], p["b1"], p["W2"], p["b2"])
        if l < 2:
            h = _norm_call(h2, cs, cq, p["gamma"], p["beta"]).reshape(NC * N, H)

    p3 = layer_ps[2]
    lin1W = jnp.stack([params["lin1"][0]["W"], params["lin1"][1]["W"]])
    lin1b = jnp.stack([params["lin1"][0]["b"], params["lin1"][1]["b"]]).reshape(NC, 1, H)
    lin2W = jnp.stack([params["lin2"][0]["W"], params["lin2"][1]["W"]])
    lin2b = jnp.stack([params["lin2"][0]["b"], params["lin2"][1]["b"]]).reshape(NC, 1, C)
    y = _pool_call(h2, cs, cq, p3["gamma"], p3["beta"], batch,
                   lin1W, lin1b, lin2W, lin2b)
    return y
